# baseline (speedup 1.0000x reference)
import functools

import jax
import jax.numpy as jnp
from jax import lax
from jax.experimental import pallas as pl
from jax.experimental.pallas import tpu as pltpu
from jax.experimental.pallas import tpu_sc as plsc


def _make_sc_kernel(num_out, num_in, num_cb, igs, flat_cb_len):
    info = plsc.get_sparse_core_info()
    nc, ns, L = info.num_cores, info.num_subcores, info.num_lanes
    nw = nc * ns
    rows_per_w = num_out // nw
    num_in_elems = num_in * igs
    iters = num_in_elems // L

    mesh = plsc.VectorSubcoreMesh(core_axis_name="c", subcore_axis_name="s")

    @functools.partial(
        pl.kernel,
        mesh=mesh,
        out_type=jax.ShapeDtypeStruct((num_out, num_in_elems), jnp.float32),
        scratch_types=[
            pltpu.VMEM((flat_cb_len,), jnp.float32),
            pltpu.VMEM((rows_per_w,), jnp.float32),
            pltpu.VMEM((num_in, num_cb), jnp.int32),
            pltpu.VMEM((num_in_elems,), jnp.float32),
        ],
        compiler_params=pltpu.CompilerParams(needs_layout_passes=False),
    )
    def k(codes_hbm, cb_hbm, scales_hbm, out_hbm, cb_v, sc_v, codes_v, out_v):
        wid = lax.axis_index("s") * nc + lax.axis_index("c")
        row0 = wid * rows_per_w
        pltpu.sync_copy(cb_hbm, cb_v)
        pltpu.sync_copy(scales_hbm.at[pl.ds(row0, rows_per_w)], sc_v)

        lane = lax.iota(jnp.int32, L)
        j_lane = lane & 7
        hi8 = lane & 8
        hi1 = lane >> 3

        def row_body(r, carry):
            row = row0 + r
            pltpu.sync_copy(codes_hbm.at[row], codes_v)
            s = plsc.load_gather(sc_v, [jnp.full((L,), r, jnp.int32)])

            def it_body(it, c2):
                base = it * 16
                codes_vec = plsc.load_gather(
                    codes_v, [it * 2 + hi1, j_lane])
                acc = jnp.zeros((L,), jnp.float32)
                for m in range(8):
                    cvec = jnp.take_along_axis(codes_vec, hi8 + m, axis=0)
                    fidx = (cvec << 3) + (j_lane + m * 2048)
                    acc = acc + plsc.load_gather(cb_v, [fidx])
                out_v[pl.ds(base, L)] = acc * s
                return c2

            lax.fori_loop(0, iters, it_body, 0, unroll=8)
            pltpu.sync_copy(out_v, out_hbm.at[row])
            return carry

        lax.fori_loop(0, rows_per_w, row_body, 0)

    return k


def kernel(codes, codebooks, scales):
    num_out, num_in, num_cb = codes.shape
    _, cb_size, ogs, igs = codebooks.shape
    flat_cb = codebooks.reshape(num_cb * cb_size * ogs * igs)
    scales1d = scales.reshape(num_out)
    k = _make_sc_kernel(num_out, num_in, num_cb, igs, flat_cb.shape[0])
    return k(codes, flat_cb, scales1d)


# speedup vs baseline: 2.0506x; 2.0506x over previous
import functools

import jax
import jax.numpy as jnp
from jax import lax
from jax.experimental import pallas as pl
from jax.experimental.pallas import tpu as pltpu
from jax.experimental.pallas import tpu_sc as plsc


def _make_sc_kernel(num_out, num_in, num_cb, igs, cb_size):
    info = plsc.get_sparse_core_info()
    nc, ns, L = info.num_cores, info.num_subcores, info.num_lanes
    nw = nc * ns
    rows_per_w = num_out // nw
    flat_cb_len = num_cb * cb_size * igs
    num_in_elems = num_in * igs
    codes_row_len = num_in * num_cb
    blocks = num_in // L

    mesh = plsc.VectorSubcoreMesh(core_axis_name="c", subcore_axis_name="s")

    @functools.partial(
        pl.kernel,
        mesh=mesh,
        out_type=jax.ShapeDtypeStruct((num_out, num_in_elems), jnp.float32),
        scratch_types=[
            pltpu.VMEM((flat_cb_len,), jnp.float32),
            pltpu.VMEM((rows_per_w,), jnp.float32),
            pltpu.VMEM((codes_row_len,), jnp.int32),
            pltpu.VMEM((num_in_elems,), jnp.float32),
        ],
        compiler_params=pltpu.CompilerParams(needs_layout_passes=False),
    )
    def k(codes_hbm, cb_hbm, scales_hbm, out_hbm, cb_v, sc_v, codes_v, out_v):
        wid = lax.axis_index("s") * nc + lax.axis_index("c")
        row0 = wid * rows_per_w
        pltpu.sync_copy(cb_hbm, cb_v)
        pltpu.sync_copy(scales_hbm.at[pl.ds(row0, rows_per_w)], sc_v)

        lane = lax.iota(jnp.int32, L)
        code_pats = [lane * num_cb + m for m in range(num_cb)]
        scat_pats = [lane * igs + j for j in range(igs)]

        def row_body(r, carry):
            row = row0 + r
            pltpu.sync_copy(codes_hbm.at[row], codes_v)
            s = plsc.load_gather(sc_v, [jnp.full((L,), r, jnp.int32)])

            def blk_body(ib, c2):
                codes_blk = codes_v.at[pl.ds(ib * (L * num_cb), L * num_cb)]
                out_blk = out_v.at[pl.ds(ib * (L * igs), L * igs)]
                cvecs = [
                    plsc.load_gather(codes_blk, [code_pats[m]])
                    for m in range(num_cb)
                ]
                for j in range(igs):
                    acc = plsc.load_gather(
                        cb_v.at[pl.ds(j * num_cb * cb_size, cb_size)],
                        [cvecs[0]])
                    for m in range(1, num_cb):
                        off = (j * num_cb + m) * cb_size
                        acc = acc + plsc.load_gather(
                            cb_v.at[pl.ds(off, cb_size)], [cvecs[m]])
                    plsc.store_scatter(out_blk, [scat_pats[j]], acc * s)
                return c2

            lax.fori_loop(0, blocks, blk_body, 0, unroll=2)
            pltpu.sync_copy(out_v, out_hbm.at[row])
            return carry

        lax.fori_loop(0, rows_per_w, row_body, 0)

    return k


def kernel(codes, codebooks, scales):
    num_out, num_in, num_cb = codes.shape
    _, cb_size, ogs, igs = codebooks.shape
    codes2d = codes.reshape(num_out, num_in * num_cb)
    flat_cb = codebooks.reshape(num_cb, cb_size, igs).transpose(2, 0, 1).reshape(-1)
    scales1d = scales.reshape(num_out)
    k = _make_sc_kernel(num_out, num_in, num_cb, igs, cb_size)
    return k(codes2d, flat_cb, scales1d)


# speedup vs baseline: 3.2989x; 1.6088x over previous
import functools

import jax
import jax.numpy as jnp
from jax import lax
from jax.experimental import pallas as pl
from jax.experimental.pallas import tpu as pltpu
from jax.experimental.pallas import tpu_sc as plsc


def _make_sc_kernel(num_out, num_in, num_cb, igs, cb_size):
    info = plsc.get_sparse_core_info()
    nc, ns, L = info.num_cores, info.num_subcores, info.num_lanes
    nw = nc * ns
    rows_per_w = num_out // nw
    flat_cb_len = num_cb * cb_size * igs
    num_in_elems = num_in * igs
    codes_row_len = num_in * num_cb
    blocks = num_in // L

    mesh = plsc.VectorSubcoreMesh(core_axis_name="c", subcore_axis_name="s")

    @functools.partial(
        pl.kernel,
        mesh=mesh,
        out_type=jax.ShapeDtypeStruct((num_out, num_in_elems), jnp.float32),
        scratch_types=[
            pltpu.VMEM((flat_cb_len,), jnp.float32),
            pltpu.VMEM((rows_per_w,), jnp.float32),
            pltpu.VMEM((codes_row_len,), jnp.int32),
            pltpu.VMEM((num_in_elems,), jnp.float32),
        ],
        compiler_params=pltpu.CompilerParams(needs_layout_passes=False),
    )
    def k(codes_hbm, cb_hbm, scales_hbm, out_hbm, cb_v, sc_v, codes_v, out_v):
        wid = lax.axis_index("s") * nc + lax.axis_index("c")
        row0 = wid * rows_per_w
        pltpu.sync_copy(cb_hbm, cb_v)
        pltpu.sync_copy(scales_hbm.at[pl.ds(row0, rows_per_w)], sc_v)

        lane = lax.iota(jnp.int32, L)
        code_pats = [lane * num_cb + m for m in range(num_cb)]
        scat_pats = [lane * igs + j for j in range(igs)]

        def row_body(r, carry):
            row = row0 + r
            pltpu.sync_copy(codes_hbm.at[row], codes_v)
            s = plsc.load_gather(sc_v, [jnp.full((L,), r, jnp.int32)])

            @plsc.parallel_loop(0, blocks, unroll=2)
            def blk_body(ib):
                codes_blk = codes_v.at[pl.ds(ib * (L * num_cb), L * num_cb)]
                out_blk = out_v.at[pl.ds(ib * (L * igs), L * igs)]
                cvecs = [
                    plsc.load_gather(codes_blk, [code_pats[m]])
                    for m in range(num_cb)
                ]
                for j in range(igs):
                    acc = plsc.load_gather(
                        cb_v.at[pl.ds(j * num_cb * cb_size, cb_size)],
                        [cvecs[0]])
                    for m in range(1, num_cb):
                        off = (j * num_cb + m) * cb_size
                        acc = acc + plsc.load_gather(
                            cb_v.at[pl.ds(off, cb_size)], [cvecs[m]])
                    plsc.store_scatter(out_blk, [scat_pats[j]], acc * s)

            pltpu.sync_copy(out_v, out_hbm.at[row])
            return carry

        lax.fori_loop(0, rows_per_w, row_body, 0)

    return k


def kernel(codes, codebooks, scales):
    num_out, num_in, num_cb = codes.shape
    _, cb_size, ogs, igs = codebooks.shape
    codes2d = codes.reshape(num_out, num_in * num_cb)
    flat_cb = codebooks.reshape(num_cb, cb_size, igs).transpose(2, 0, 1).reshape(-1)
    scales1d = scales.reshape(num_out)
    k = _make_sc_kernel(num_out, num_in, num_cb, igs, cb_size)
    return k(codes2d, flat_cb, scales1d)
